# Initial kernel scaffold; baseline (speedup 1.0000x reference)
import jax
import jax.numpy as jnp
from jax.experimental import pallas as pl


def kernel(x, edge_index, W1, b1, W2, b2, W_ih, W_hh, b_ih, b_hh, W_fc, b_fc):
    raise NotImplementedError("write your pallas kernel here")


# speedup vs baseline: 6.2539x; 6.2539x over previous
import jax
import jax.numpy as jnp
from jax import lax
from jax.experimental import pallas as pl
from jax.experimental.pallas import tpu as pltpu
from jax.experimental.pallas import tpu_sc as plsc

N = 10000
E = 320000
IN_FEAT = 128
HID = 32
GHID = 3 * HID
OUT_FEAT = 3

NC, NS = 2, 16
NW = NC * NS
CHUNK = 128
CPT = -(-E // (NW * CHUNK))
E_PAD = NW * CPT * CHUNK
N_PAD = 10240
RPS = N_PAD // NS

import functools


@functools.cache
def _agg_kernel():
    mesh = plsc.VectorSubcoreMesh(core_axis_name="c", subcore_axis_name="s",
                                  num_cores=NC, num_subcores=NS)
    return pl.kernel(
        _agg_body,
        out_type=jax.ShapeDtypeStruct((NC, N_PAD, HID), jnp.float32),
        mesh=mesh,
        scratch_types=[
            pltpu.VMEM((CPT, CHUNK), jnp.int32),
            pltpu.VMEM((CPT, CHUNK), jnp.int32),
            pltpu.VMEM((CHUNK, HID), jnp.float32),
            pltpu.VMEM_SHARED((N_PAD, HID), jnp.float32),
            pltpu.SemaphoreType.DMA,
        ],
        compiler_params=pltpu.CompilerParams(use_tc_tiling_on_sc=False),
    )


def _agg(y, src3, dst3, zinit):
    return _agg_kernel()(y, src3, dst3, zinit)


def _agg_body(y_hbm, src_hbm, dst_hbm, zinit_hbm, out_hbm,
              srcv, dstv, rows_v, acc_sh, sem):
    c = lax.axis_index("c")
    s = lax.axis_index("s")
    w = c * NS + s
    pltpu.sync_copy(zinit_hbm, acc_sh.at[pl.ds(s * RPS, RPS)])
    pltpu.sync_copy(src_hbm.at[w], srcv)
    pltpu.sync_copy(dst_hbm.at[w], dstv)
    plsc.subcore_barrier()

    def chunk(j, carry):
        pltpu.async_copy(y_hbm.at[srcv.at[j]], rows_v, sem).wait()
        pltpu.sync_copy(rows_v, acc_sh.at[dstv.at[j]], add=True)
        return carry

    lax.fori_loop(0, CPT, chunk, 0)
    plsc.subcore_barrier()
    pltpu.sync_copy(acc_sh.at[pl.ds(s * RPS, RPS)],
                    out_hbm.at[c, pl.ds(s * RPS, RPS)])


_B = 1280
_DOT = dict(preferred_element_type=jnp.float32, precision=lax.Precision.HIGHEST)


def _dinv(degp_ref):
    deg = degp_ref[0, :, 0:1] + degp_ref[1, :, 0:1] + 1.0
    return lax.rsqrt(deg)


def _y1_body(x_ref, w1_ref, degp_ref, y1_ref):
    xw = lax.dot_general(x_ref[...], w1_ref[...], (((1,), (0,)), ((), ())), **_DOT)
    y1_ref[...] = xw * _dinv(degp_ref)


_y1_call = pl.pallas_call(
    _y1_body,
    grid=(N_PAD // _B,),
    in_specs=[
        pl.BlockSpec((_B, IN_FEAT), lambda i: (i, 0)),
        pl.BlockSpec((IN_FEAT, HID), lambda i: (0, 0)),
        pl.BlockSpec((NC, _B, HID), lambda i: (0, i, 0)),
    ],
    out_specs=pl.BlockSpec((_B, HID), lambda i: (i, 0)),
    out_shape=jax.ShapeDtypeStruct((N_PAD, HID), jnp.float32),
)


def _l2_body(p_ref, y1_ref, degp_ref, b1_ref, w2_ref, y2_ref):
    dinv = _dinv(degp_ref)
    h1 = jnp.maximum(dinv * (p_ref[0] + p_ref[1] + y1_ref[...]) + b1_ref[...], 0.0)
    t = lax.dot_general(h1, w2_ref[...], (((1,), (0,)), ((), ())), **_DOT)
    y2_ref[...] = t * dinv


_l2_call = pl.pallas_call(
    _l2_body,
    grid=(N_PAD // _B,),
    in_specs=[
        pl.BlockSpec((NC, _B, HID), lambda i: (0, i, 0)),
        pl.BlockSpec((_B, HID), lambda i: (i, 0)),
        pl.BlockSpec((NC, _B, HID), lambda i: (0, i, 0)),
        pl.BlockSpec((1, HID), lambda i: (0, 0)),
        pl.BlockSpec((HID, HID), lambda i: (0, 0)),
    ],
    out_specs=pl.BlockSpec((_B, HID), lambda i: (i, 0)),
    out_shape=jax.ShapeDtypeStruct((N_PAD, HID), jnp.float32),
)


def _gi_body(p_ref, y2_ref, degp_ref, b2_ref, wih_ref, bih_ref, gi_ref):
    dinv = _dinv(degp_ref)
    h2 = jnp.maximum(dinv * (p_ref[0] + p_ref[1] + y2_ref[...]) + b2_ref[...], 0.0)
    gi_ref[...] = lax.dot_general(h2, wih_ref[...], (((1,), (0,)), ((), ())),
                                  **_DOT) + bih_ref[...]


_gi_call = pl.pallas_call(
    _gi_body,
    grid=(N_PAD // _B,),
    in_specs=[
        pl.BlockSpec((NC, _B, HID), lambda i: (0, i, 0)),
        pl.BlockSpec((_B, HID), lambda i: (i, 0)),
        pl.BlockSpec((NC, _B, HID), lambda i: (0, i, 0)),
        pl.BlockSpec((1, HID), lambda i: (0, 0)),
        pl.BlockSpec((HID, GHID), lambda i: (0, 0)),
        pl.BlockSpec((1, GHID), lambda i: (0, 0)),
    ],
    out_specs=pl.BlockSpec((_B, GHID), lambda i: (i, 0)),
    out_shape=jax.ShapeDtypeStruct((N_PAD, GHID), jnp.float32),
)


def _sigmoid(a):
    return 1.0 / (1.0 + jnp.exp(-a))


def _scan_body(gi_ref, whh_ref, bhh_ref, ys_ref):
    whh = whh_ref[...]
    bhh = bhh_ref[...]

    def step(t, h):
        gi = gi_ref[pl.ds(t, 1), :]
        gh = lax.dot_general(h, whh, (((1,), (0,)), ((), ())), **_DOT) + bhh
        r = _sigmoid(gi[:, 0:HID] + gh[:, 0:HID])
        z = _sigmoid(gi[:, HID:2 * HID] + gh[:, HID:2 * HID])
        n = jnp.tanh(gi[:, 2 * HID:] + r * gh[:, 2 * HID:])
        h_new = (1.0 - z) * n + z * h
        ys_ref[pl.ds(t, 1), :] = h_new
        return h_new

    lax.fori_loop(0, N, step, jnp.zeros((1, HID), jnp.float32))


_scan_call = pl.pallas_call(
    _scan_body,
    in_specs=[
        pl.BlockSpec((N_PAD, GHID), lambda: (0, 0)),
        pl.BlockSpec((HID, GHID), lambda: (0, 0)),
        pl.BlockSpec((1, GHID), lambda: (0, 0)),
    ],
    out_specs=pl.BlockSpec((N, HID), lambda: (0, 0)),
    out_shape=jax.ShapeDtypeStruct((N, HID), jnp.float32),
)

_BF = 1000


def _fc_body(ys_ref, wfc_ref, bfc_ref, o_ref):
    o_ref[...] = lax.dot_general(ys_ref[...], wfc_ref[...], (((1,), (0,)), ((), ())),
                                 **_DOT) + bfc_ref[...]


_fc_call = pl.pallas_call(
    _fc_body,
    grid=(N // _BF,),
    in_specs=[
        pl.BlockSpec((_BF, HID), lambda i: (i, 0)),
        pl.BlockSpec((HID, OUT_FEAT), lambda i: (0, 0)),
        pl.BlockSpec((1, OUT_FEAT), lambda i: (0, 0)),
    ],
    out_specs=pl.BlockSpec((_BF, OUT_FEAT), lambda i: (i, 0)),
    out_shape=jax.ShapeDtypeStruct((N, OUT_FEAT), jnp.float32),
)


def kernel(x, edge_index, W1, b1, W2, b2, W_ih, W_hh, b_ih, b_hh, W_fc, b_fc):
    f32 = jnp.float32
    x_pad = jnp.pad(x.astype(f32), ((0, N_PAD - N), (0, 0)))
    pad = jnp.full((E_PAD - E,), N, jnp.int32)
    src3 = jnp.concatenate([edge_index[0].astype(jnp.int32), pad]).reshape(NW, CPT, CHUNK)
    dst3 = jnp.concatenate([edge_index[1].astype(jnp.int32), pad]).reshape(NW, CPT, CHUNK)
    zinit = jnp.zeros((RPS, HID), f32)
    ones_y = jnp.ones((N_PAD, HID), f32)

    degp = _agg(ones_y, src3, dst3, zinit)
    y1 = _y1_call(x_pad, W1, degp)
    p1 = _agg(y1, src3, dst3, zinit)
    y2 = _l2_call(p1, y1, degp, b1.reshape(1, HID), W2)
    p2 = _agg(y2, src3, dst3, zinit)
    gi = _gi_call(p2, y2, degp, b2.reshape(1, HID), W_ih.T, b_ih.reshape(1, GHID))
    ys = _scan_call(gi, W_hh.T, b_hh.reshape(1, GHID))
    out = _fc_call(ys, W_fc.T, b_fc.reshape(1, OUT_FEAT))
    return out.reshape(1, N, OUT_FEAT)
